# Initial kernel scaffold; baseline (speedup 1.0000x reference)
import jax
import jax.numpy as jnp
from jax.experimental import pallas as pl


def kernel(x, edge_index, pseudo, W1, root1, b1, W2, root2, b2):
    raise NotImplementedError("write your pallas kernel here")


# speedup vs baseline: 6.3268x; 6.3268x over previous
import functools

import jax
import jax.numpy as jnp
from jax import lax
from jax.experimental import pallas as pl
from jax.experimental.pallas import tpu as pltpu
from jax.experimental.pallas import tpu_sc as plsc

N = 10000
E = 160000
D_IN = 1433

_info = plsc.get_sparse_core_info()
NC, NS = _info.num_cores, _info.num_subcores
NW = NC * NS

NPAD = 10112
ROWS_PER_TILE = NPAD // NS
EPW = 5120
EPAD = NW * EPW
CHUNK = 1024
IDX_ROWS = CHUNK // 128
NCHUNK = EPW // CHUNK


def _mm1_body(x_ref, w_ref, y01_ref, yroot_ref):
    y = jnp.dot(x_ref[...], w_ref[...], preferred_element_type=jnp.float32)
    y01_ref[...] = y[:, :32]
    yroot_ref[...] = y[:, 32:48]


def _mm1(x, wcat):
    bm = 512
    grid = (pl.cdiv(NPAD, bm),)
    return pl.pallas_call(
        _mm1_body,
        grid=grid,
        in_specs=[
            pl.BlockSpec((bm, D_IN), lambda i: (i, 0)),
            pl.BlockSpec((D_IN, 48), lambda i: (0, 0)),
        ],
        out_specs=[
            pl.BlockSpec((bm, 32), lambda i: (i, 0)),
            pl.BlockSpec((bm, 16), lambda i: (i, 0)),
        ],
        out_shape=[
            jax.ShapeDtypeStruct((NPAD, 32), jnp.float32),
            jax.ShapeDtypeStruct((NPAD, 16), jnp.float32),
        ],
    )(x, wcat)


def _edge_agg_body(with_deg, ytab, src2d, dst2d, u16, zrows, ones_hbm,
                   agg_out, deg_out, sidx, didx, uv, gv, mv, ones_v,
                   agg_sh, deg_sh, sem_g, sem_s):
    c = lax.axis_index("c")
    s = lax.axis_index("s")
    wid = s * NC + c
    r0_tile = s * ROWS_PER_TILE

    pltpu.sync_copy(zrows.at[pl.ds(r0_tile, ROWS_PER_TILE)],
                    agg_sh.at[pl.ds(r0_tile, ROWS_PER_TILE)])
    if with_deg:
        pltpu.sync_copy(zrows.at[pl.ds(r0_tile, ROWS_PER_TILE)],
                        deg_sh.at[pl.ds(r0_tile, ROWS_PER_TILE)])
        pltpu.sync_copy(ones_hbm, ones_v)
    plsc.subcore_barrier()

    idx_row0 = wid * (EPW // 128)
    e0 = wid * EPW
    for ch in range(NCHUNK):
        r0 = idx_row0 + ch * IDX_ROWS
        pltpu.sync_copy(src2d.at[pl.ds(r0, IDX_ROWS)], sidx)
        pltpu.sync_copy(dst2d.at[pl.ds(r0, IDX_ROWS)], didx)
        pltpu.sync_copy(u16.at[pl.ds(e0 + ch * CHUNK, CHUNK)], uv)
        gathers = [
            pltpu.async_copy(ytab.at[sidx.at[j]],
                             gv.at[pl.ds(j * 128, 128)], sem_g)
            for j in range(IDX_ROWS)
        ]
        for h in gathers:
            h.wait()

        def body(e, carry):
            g0 = gv[e, pl.ds(0, 16)]
            g1 = gv[e, pl.ds(16, 16)]
            uu = uv[e, :]
            mv[e, :] = g0 + uu * (g1 - g0)
            return carry

        lax.fori_loop(0, CHUNK, body, 0, unroll=4)

        scatters = []
        for j in range(IDX_ROWS):
            scatters.append(
                pltpu.async_copy(mv.at[pl.ds(j * 128, 128)],
                                 agg_sh.at[didx.at[j]], sem_s, add=True))
            if with_deg:
                scatters.append(
                    pltpu.async_copy(ones_v, deg_sh.at[didx.at[j]],
                                     sem_s, add=True))
        for h in scatters:
            h.wait()

    plsc.subcore_barrier()
    pltpu.sync_copy(agg_sh.at[pl.ds(r0_tile, ROWS_PER_TILE)],
                    agg_out.at[c, pl.ds(r0_tile, ROWS_PER_TILE)])
    if with_deg:
        pltpu.sync_copy(deg_sh.at[pl.ds(r0_tile, ROWS_PER_TILE)],
                        deg_out.at[c, pl.ds(r0_tile, ROWS_PER_TILE)])


def _edge_agg(with_deg):
    mesh = plsc.VectorSubcoreMesh(core_axis_name="c", subcore_axis_name="s")
    out_type = [
        jax.ShapeDtypeStruct((NC, NPAD, 16), jnp.float32),
        jax.ShapeDtypeStruct((NC, NPAD, 16), jnp.float32),
    ]
    scratch = [
        pltpu.VMEM((IDX_ROWS, 128), jnp.int32),
        pltpu.VMEM((IDX_ROWS, 128), jnp.int32),
        pltpu.VMEM((CHUNK, 16), jnp.float32),
        pltpu.VMEM((CHUNK, 32), jnp.float32),
        pltpu.VMEM((CHUNK, 16), jnp.float32),
        pltpu.VMEM((128, 16), jnp.float32),
        pltpu.VMEM_SHARED((NPAD, 16), jnp.float32),
        pltpu.VMEM_SHARED((NPAD, 16), jnp.float32),
        pltpu.SemaphoreType.DMA,
        pltpu.SemaphoreType.DMA,
    ]
    return pl.kernel(
        functools.partial(_edge_agg_body, with_deg),
        out_type=out_type,
        mesh=mesh,
        scratch_types=scratch,
        compiler_params=pltpu.CompilerParams(use_tc_tiling_on_sc=False),
    )


def _mid_body(agg_ref, deg_ref, yroot_ref, b1_ref, w2_ref, z01_ref, zroot_ref):
    a = agg_ref[0] + agg_ref[1]
    d = deg_ref[0] + deg_ref[1]
    rdeg = 1.0 / jnp.maximum(d, 1.0)
    pre = a * rdeg + yroot_ref[...] + b1_ref[...]
    h = jnp.where(pre > 0, pre, jnp.exp(jnp.minimum(pre, 0.0)) - 1.0)
    z = jnp.dot(h, w2_ref[...], preferred_element_type=jnp.float32)
    z01_ref[...] = z[:, :32]
    zroot_ref[...] = z[:, 32:48]


def _mid(agg, deg, yroot, b1, wcat2):
    br = 2528
    grid = (NPAD // br,)
    return pl.pallas_call(
        _mid_body,
        grid=grid,
        in_specs=[
            pl.BlockSpec((2, br, 16), lambda i: (0, i, 0)),
            pl.BlockSpec((2, br, 16), lambda i: (0, i, 0)),
            pl.BlockSpec((br, 16), lambda i: (i, 0)),
            pl.BlockSpec((1, 16), lambda i: (0, 0)),
            pl.BlockSpec((16, 48), lambda i: (0, 0)),
        ],
        out_specs=[
            pl.BlockSpec((br, 32), lambda i: (i, 0)),
            pl.BlockSpec((br, 16), lambda i: (i, 0)),
        ],
        out_shape=[
            jax.ShapeDtypeStruct((NPAD, 32), jnp.float32),
            jax.ShapeDtypeStruct((NPAD, 16), jnp.float32),
        ],
    )(agg, deg, yroot, b1, wcat2)


def _fin_body(agg_ref, deg_ref, zroot_ref, b2_ref, out_ref):
    a = agg_ref[0] + agg_ref[1]
    d = deg_ref[0] + deg_ref[1]
    rdeg = 1.0 / jnp.maximum(d, 1.0)
    s = a * rdeg + zroot_ref[...] + b2_ref[...]
    col = lax.broadcasted_iota(jnp.int32, s.shape, 1)
    valid = col < 7
    sm = jnp.where(valid, s, -jnp.inf)
    m = jnp.max(sm, axis=1, keepdims=True)
    e = jnp.where(valid, jnp.exp(s - m), 0.0)
    tot = jnp.sum(e, axis=1, keepdims=True)
    out_ref[...] = s - m - jnp.log(tot)


def _fin(agg, deg, zroot, b2):
    br = 2528
    grid = (NPAD // br,)
    return pl.pallas_call(
        _fin_body,
        grid=grid,
        in_specs=[
            pl.BlockSpec((2, br, 16), lambda i: (0, i, 0)),
            pl.BlockSpec((2, br, 16), lambda i: (0, i, 0)),
            pl.BlockSpec((br, 16), lambda i: (i, 0)),
            pl.BlockSpec((1, 16), lambda i: (0, 0)),
        ],
        out_specs=pl.BlockSpec((br, 16), lambda i: (i, 0)),
        out_shape=jax.ShapeDtypeStruct((NPAD, 16), jnp.float32),
    )(agg, deg, zroot, b2)


def kernel(x, edge_index, pseudo, W1, root1, b1, W2, root2, b2):
    f32 = jnp.float32
    src = edge_index[0].astype(jnp.int32)
    dst = edge_index[1].astype(jnp.int32)
    u = jnp.clip(pseudo[:, 0], 0.0, 1.0)

    npad_e = EPAD - E
    srcp = jnp.concatenate([src, jnp.zeros((npad_e,), jnp.int32)])
    dstp = jnp.concatenate([dst, jnp.full((npad_e,), N, jnp.int32)])
    up = jnp.concatenate([u, jnp.zeros((npad_e,), f32)])
    src2d = srcp.reshape(EPAD // 128, 128)
    dst2d = dstp.reshape(EPAD // 128, 128)
    u16 = jnp.broadcast_to(up[:, None], (EPAD, 16))

    zrows = jnp.zeros((NPAD, 16), f32)
    ones_hbm = jnp.ones((128, 16), f32)

    wcat1 = jnp.concatenate([W1[0], W1[1], root1], axis=1)
    pad7 = lambda w: jnp.pad(w, ((0, 0), (0, 16 - w.shape[1])))
    wcat2 = jnp.concatenate([pad7(W2[0]), pad7(W2[1]), pad7(root2)], axis=1)
    b1r = b1.reshape(1, 16)
    b2r = jnp.pad(b2, (0, 9)).reshape(1, 16)

    y01, yroot = _mm1(x, wcat1)
    agg1, deg = _edge_agg(True)(y01, src2d, dst2d, u16, zrows, ones_hbm)
    z01, zroot = _mid(agg1, deg, yroot, b1r, wcat2)
    agg2, _ = _edge_agg(False)(z01, src2d, dst2d, u16, zrows, ones_hbm)
    out = _fin(agg2, deg, zroot, b2r)
    return out[:N, :7]


# speedup vs baseline: 10.9934x; 1.7376x over previous
import functools

import jax
import jax.numpy as jnp
from jax import lax
from jax.experimental import pallas as pl
from jax.experimental.pallas import tpu as pltpu
from jax.experimental.pallas import tpu_sc as plsc

N = 10000
E = 160000
D_IN = 1433

_info = plsc.get_sparse_core_info()
NC, NS = _info.num_cores, _info.num_subcores
NW = NC * NS

NPAD = 10112
ROWS_PER_TILE = NPAD // NS
EPW = 5120
EPAD = NW * EPW
CHUNK = 512
IDX_ROWS = CHUNK // 128
NCHUNK = EPW // CHUNK


def _mm1_body(x_ref, w_ref, y01_ref, yroot_ref):
    y = jnp.dot(x_ref[...], w_ref[...], preferred_element_type=jnp.float32)
    y01_ref[...] = y[:, :32]
    yroot_ref[...] = y[:, 32:48]


def _mm1(x, wcat):
    bm = 512
    grid = (pl.cdiv(NPAD, bm),)
    return pl.pallas_call(
        _mm1_body,
        grid=grid,
        in_specs=[
            pl.BlockSpec((bm, D_IN), lambda i: (i, 0)),
            pl.BlockSpec((D_IN, 48), lambda i: (0, 0)),
        ],
        out_specs=[
            pl.BlockSpec((bm, 32), lambda i: (i, 0)),
            pl.BlockSpec((bm, 16), lambda i: (i, 0)),
        ],
        out_shape=[
            jax.ShapeDtypeStruct((NPAD, 32), jnp.float32),
            jax.ShapeDtypeStruct((NPAD, 16), jnp.float32),
        ],
    )(x, wcat)


def _edge_agg_body(with_deg, ytab, src2d, dst2d, u1d, zrows,
                   agg_out, sidx, didx, uv, gv, mv, agg_sh,
                   sem_l, sem_g0, sem_g1, sem_s0, sem_s1):
    w = 32 if with_deg else 16
    c = lax.axis_index("c")
    s = lax.axis_index("s")
    wid = s * NC + c
    r0_tile = s * ROWS_PER_TILE
    idx_row0 = wid * (EPW // 128)
    e0 = wid * EPW
    sem_g = (sem_g0, sem_g1)
    sem_s = (sem_s0, sem_s1)

    def fire_loads(ch, b):
        r0 = idx_row0 + ch * IDX_ROWS
        return [
            pltpu.async_copy(src2d.at[pl.ds(r0, IDX_ROWS)], sidx.at[b], sem_l),
            pltpu.async_copy(dst2d.at[pl.ds(r0, IDX_ROWS)],
                             didx.at[ch % 4], sem_l),
            pltpu.async_copy(u1d.at[pl.ds(e0 + ch * CHUNK, CHUNK)],
                             uv.at[b], sem_l),
        ]

    def fire_gathers(ch, b):
        return [
            pltpu.async_copy(ytab.at[sidx.at[b, j]],
                             gv.at[b, pl.ds(j * 128, 128)], sem_g[b])
            for j in range(IDX_ROWS)
        ]

    def fire_scatters(ch, b):
        return [
            pltpu.async_copy(mv.at[b, pl.ds(j * 128, 128)],
                             agg_sh.at[didx.at[ch % 4, j]], sem_s[b],
                             add=True)
            for j in range(IDX_ROWS)
        ]

    def compute(b):
        def grp(g, carry):
            base = g * 16
            uvec = uv[b, pl.ds(base, 16)]
            for k in range(16):
                uu = lax.gather(
                    uvec, jnp.full((16, 1), k, jnp.int32),
                    lax.GatherDimensionNumbers(
                        offset_dims=(), collapsed_slice_dims=(0,),
                        start_index_map=(0,)),
                    slice_sizes=(1,),
                    mode=lax.GatherScatterMode.PROMISE_IN_BOUNDS)
                g0 = gv[b, base + k, pl.ds(0, 16)]
                g1 = gv[b, base + k, pl.ds(16, 16)]
                mv[b, base + k, pl.ds(0, 16)] = g0 + uu * (g1 - g0)
            return carry
        lax.fori_loop(0, CHUNK // 16, grp, 0)

    lh = {0: fire_loads(0, 0), 1: fire_loads(1, 1)}

    pltpu.sync_copy(zrows.at[pl.ds(r0_tile, ROWS_PER_TILE)],
                    agg_sh.at[pl.ds(r0_tile, ROWS_PER_TILE)])
    if with_deg:
        ones = jnp.ones((16,), jnp.float32)

        def oinit(e, carry):
            mv[0, e, pl.ds(16, 16)] = ones
            mv[1, e, pl.ds(16, 16)] = ones
            return carry
        lax.fori_loop(0, CHUNK, oinit, 0)
    plsc.subcore_barrier()

    for h in lh[0]:
        h.wait()
    gh = {0: fire_gathers(0, 0)}
    sh = {}
    for ch in range(NCHUNK):
        b = ch & 1
        if ch + 1 < NCHUNK:
            for h in lh[ch + 1]:
                h.wait()
            gh[ch + 1] = fire_gathers(ch + 1, 1 - b)
        for h in gh[ch]:
            h.wait()
        if ch >= 2:
            for h in sh[ch - 2]:
                h.wait()
        compute(b)
        sh[ch] = fire_scatters(ch, b)
        if ch + 2 < NCHUNK:
            lh[ch + 2] = fire_loads(ch + 2, b)
    for h in sh[NCHUNK - 2] + sh[NCHUNK - 1]:
        h.wait()

    plsc.subcore_barrier()
    pltpu.sync_copy(agg_sh.at[pl.ds(r0_tile, ROWS_PER_TILE)],
                    agg_out.at[c, pl.ds(r0_tile, ROWS_PER_TILE)])


def _edge_agg(with_deg):
    w = 32 if with_deg else 16
    mesh = plsc.VectorSubcoreMesh(core_axis_name="c", subcore_axis_name="s")
    out_type = jax.ShapeDtypeStruct((NC, NPAD, w), jnp.float32)
    scratch = [
        pltpu.VMEM((2, IDX_ROWS, 128), jnp.int32),
        pltpu.VMEM((4, IDX_ROWS, 128), jnp.int32),
        pltpu.VMEM((2, CHUNK), jnp.float32),
        pltpu.VMEM((2, CHUNK, 32), jnp.float32),
        pltpu.VMEM((2, CHUNK, w), jnp.float32),
        pltpu.VMEM_SHARED((NPAD, w), jnp.float32),
        pltpu.SemaphoreType.DMA,
        pltpu.SemaphoreType.DMA,
        pltpu.SemaphoreType.DMA,
        pltpu.SemaphoreType.DMA,
        pltpu.SemaphoreType.DMA,
    ]
    return pl.kernel(
        functools.partial(_edge_agg_body, with_deg),
        out_type=out_type,
        mesh=mesh,
        scratch_types=scratch,
        compiler_params=pltpu.CompilerParams(use_tc_tiling_on_sc=False),
    )


def _mid_body(agg_ref, yroot_ref, b1_ref, w2_ref, z01_ref, zroot_ref, rdeg_ref):
    a = agg_ref[0, :, :16] + agg_ref[1, :, :16]
    d = agg_ref[0, :, 16:32] + agg_ref[1, :, 16:32]
    rdeg = 1.0 / jnp.maximum(d, 1.0)
    rdeg_ref[...] = rdeg
    pre = a * rdeg + yroot_ref[...] + b1_ref[...]
    h = jnp.where(pre > 0, pre, jnp.exp(jnp.minimum(pre, 0.0)) - 1.0)
    z = jnp.dot(h, w2_ref[...], preferred_element_type=jnp.float32)
    z01_ref[...] = z[:, :32]
    zroot_ref[...] = z[:, 32:48]


def _mid(agg, yroot, b1, wcat2):
    br = 2528
    grid = (NPAD // br,)
    return pl.pallas_call(
        _mid_body,
        grid=grid,
        in_specs=[
            pl.BlockSpec((2, br, 32), lambda i: (0, i, 0)),
            pl.BlockSpec((br, 16), lambda i: (i, 0)),
            pl.BlockSpec((1, 16), lambda i: (0, 0)),
            pl.BlockSpec((16, 48), lambda i: (0, 0)),
        ],
        out_specs=[
            pl.BlockSpec((br, 32), lambda i: (i, 0)),
            pl.BlockSpec((br, 16), lambda i: (i, 0)),
            pl.BlockSpec((br, 16), lambda i: (i, 0)),
        ],
        out_shape=[
            jax.ShapeDtypeStruct((NPAD, 32), jnp.float32),
            jax.ShapeDtypeStruct((NPAD, 16), jnp.float32),
            jax.ShapeDtypeStruct((NPAD, 16), jnp.float32),
        ],
    )(agg, yroot, b1, wcat2)


def _fin_body(agg_ref, rdeg_ref, zroot_ref, b2_ref, out_ref):
    a = agg_ref[0] + agg_ref[1]
    s = a * rdeg_ref[...] + zroot_ref[...] + b2_ref[...]
    col = lax.broadcasted_iota(jnp.int32, s.shape, 1)
    valid = col < 7
    sm = jnp.where(valid, s, -jnp.inf)
    m = jnp.max(sm, axis=1, keepdims=True)
    e = jnp.where(valid, jnp.exp(s - m), 0.0)
    tot = jnp.sum(e, axis=1, keepdims=True)
    out_ref[...] = s - m - jnp.log(tot)


def _fin(agg, rdeg, zroot, b2):
    br = 2528
    grid = (NPAD // br,)
    return pl.pallas_call(
        _fin_body,
        grid=grid,
        in_specs=[
            pl.BlockSpec((2, br, 16), lambda i: (0, i, 0)),
            pl.BlockSpec((br, 16), lambda i: (i, 0)),
            pl.BlockSpec((br, 16), lambda i: (i, 0)),
            pl.BlockSpec((1, 16), lambda i: (0, 0)),
        ],
        out_specs=pl.BlockSpec((br, 16), lambda i: (i, 0)),
        out_shape=jax.ShapeDtypeStruct((NPAD, 16), jnp.float32),
    )(agg, rdeg, zroot, b2)


def kernel(x, edge_index, pseudo, W1, root1, b1, W2, root2, b2):
    f32 = jnp.float32
    src = edge_index[0].astype(jnp.int32)
    dst = edge_index[1].astype(jnp.int32)
    u = jnp.clip(pseudo[:, 0], 0.0, 1.0)

    npad_e = EPAD - E
    srcp = jnp.concatenate([src, jnp.zeros((npad_e,), jnp.int32)])
    dstp = jnp.concatenate([dst, jnp.full((npad_e,), N, jnp.int32)])
    up = jnp.concatenate([u, jnp.zeros((npad_e,), f32)])
    src2d = srcp.reshape(EPAD // 128, 128)
    dst2d = dstp.reshape(EPAD // 128, 128)

    zrows32 = jnp.zeros((NPAD, 32), f32)
    zrows16 = jnp.zeros((NPAD, 16), f32)

    wcat1 = jnp.concatenate([W1[0], W1[1], root1], axis=1)
    pad7 = lambda w: jnp.pad(w, ((0, 0), (0, 16 - w.shape[1])))
    wcat2 = jnp.concatenate([pad7(W2[0]), pad7(W2[1]), pad7(root2)], axis=1)
    b1r = b1.reshape(1, 16)
    b2r = jnp.pad(b2, (0, 9)).reshape(1, 16)

    y01, yroot = _mm1(x, wcat1)
    agg1 = _edge_agg(True)(y01, src2d, dst2d, up, zrows32)
    z01, zroot, rdeg = _mid(agg1, yroot, b1r, wcat2)
    agg2 = _edge_agg(False)(z01, src2d, dst2d, up, zrows16)
    out = _fin(agg2, rdeg, zroot, b2r)
    return out[:N, :7]
